# Initial kernel scaffold; baseline (speedup 1.0000x reference)
import jax
import jax.numpy as jnp
from jax.experimental import pallas as pl


def kernel(nodes, neighbor_nodes, emb, W1, b1, W2, b2, W3, b3):
    raise NotImplementedError("write your pallas kernel here")


# speedup vs baseline: 1.8520x; 1.8520x over previous
import functools

import jax
import jax.numpy as jnp
from jax import lax
from jax.experimental import pallas as pl
from jax.experimental.pallas import tpu as pltpu
from jax.experimental.pallas import tpu_sc as plsc

_N = 10000
_K = 32
_D = 128
_NPAD = 10240
_BN = 80
_NW = 32
_CHUNK_ROWS = 4
_CHUNK = _CHUNK_ROWS * 128


def _sc_gather(emb, idx_nbr2, idx_u2):
    mesh = plsc.VectorSubcoreMesh(core_axis_name="c", subcore_axis_name="s")
    nchunks = _NPAD // _CHUNK
    rows_per_w = _NPAD // 128

    @functools.partial(
        pl.kernel,
        out_type=(
            jax.ShapeDtypeStruct((_K * _NPAD, _D), jnp.float32),
            jax.ShapeDtypeStruct((_NPAD, _D), jnp.float32),
        ),
        mesh=mesh,
        scratch_types=[
            pltpu.VMEM((_CHUNK_ROWS, 128), jnp.int32),
            pltpu.VMEM((_CHUNK, _D), jnp.float32),
            pltpu.SemaphoreType.DMA,
        ],
    )
    def gather_kernel(emb_hbm, idxn_hbm, idxu_hbm, nbr_out, u_out,
                      idx_v, rows_v, sem):
        wid = lax.axis_index("s") * 2 + lax.axis_index("c")
        base_row = wid * rows_per_w
        base_emb = wid * _NPAD

        def chunk(c, carry):
            r0 = base_row + c * _CHUNK_ROWS
            pltpu.sync_copy(idxn_hbm.at[pl.ds(r0, _CHUNK_ROWS)], idx_v)
            cps = [
                pltpu.async_copy(
                    emb_hbm.at[idx_v.at[r]],
                    rows_v.at[pl.ds(r * 128, 128)],
                    sem,
                )
                for r in range(_CHUNK_ROWS)
            ]
            for cp in cps:
                cp.wait()
            pltpu.sync_copy(
                rows_v, nbr_out.at[pl.ds(base_emb + c * _CHUNK, _CHUNK)]
            )
            return carry

        lax.fori_loop(0, nchunks, chunk, 0)

        @pl.when(wid < 16)
        def _():
            def uchunk(c, carry):
                r0 = wid * 5 + c
                pltpu.sync_copy(idxu_hbm.at[pl.ds(r0, 1)],
                                idx_v.at[pl.ds(0, 1)])
                cp = pltpu.async_copy(
                    emb_hbm.at[idx_v.at[0]], rows_v.at[pl.ds(0, 128)], sem
                )
                cp.wait()
                pltpu.sync_copy(rows_v.at[pl.ds(0, 128)],
                                u_out.at[pl.ds(r0 * 128, 128)])
                return carry

            lax.fori_loop(0, 5, uchunk, 0)

    return gather_kernel(emb, idx_nbr2, idx_u2)


def _tc_body(nbr_ref, u_ref, w1a_ref, w1b_ref, b1_ref, w2_ref, b2_ref,
             w3_ref, out_ref):
    ub = u_ref[...]
    t = jnp.dot(ub, w1b_ref[...],
                preferred_element_type=jnp.float32) + b1_ref[...]
    nbr = nbr_ref[...]
    nbr_flat = nbr.reshape(_K * _BN, _D)
    h = jnp.dot(nbr_flat, w1a_ref[...], preferred_element_type=jnp.float32)
    h = h + jnp.broadcast_to(t[None], (_K, _BN, _D)).reshape(_K * _BN, _D)
    h = jnp.maximum(h, 0.0)
    h2 = jnp.dot(h, w2_ref[...], preferred_element_type=jnp.float32)
    h2 = jnp.maximum(h2 + b2_ref[...], 0.0)
    h2_3 = h2.reshape(_K, _BN, _D)
    w3v = w3_ref[...]
    s = [jnp.sum(h2_3[k] * w3v, axis=1, keepdims=True) for k in range(_K)]
    m = functools.reduce(jnp.maximum, s)
    e = [jnp.exp(sk - m) for sk in s]
    den = functools.reduce(lambda a, b: a + b, e)
    acc = e[0] * nbr[0]
    for k in range(1, _K):
        acc = acc + e[k] * nbr[k]
    out_ref[...] = acc / den


def _tc_mlp(nbr_t, u, w1a, w1b, b1, w2, b2, w3r, interpret=False):
    grid = _N // _BN
    return pl.pallas_call(
        _tc_body,
        grid=(grid,),
        in_specs=[
            pl.BlockSpec((_K, _BN, _D), lambda i: (0, i, 0)),
            pl.BlockSpec((_BN, _D), lambda i: (i, 0)),
            pl.BlockSpec((_D, _D), lambda i: (0, 0)),
            pl.BlockSpec((_D, _D), lambda i: (0, 0)),
            pl.BlockSpec((1, _D), lambda i: (0, 0)),
            pl.BlockSpec((_D, _D), lambda i: (0, 0)),
            pl.BlockSpec((1, _D), lambda i: (0, 0)),
            pl.BlockSpec((1, _D), lambda i: (0, 0)),
        ],
        out_specs=pl.BlockSpec((_BN, _D), lambda i: (i, 0)),
        out_shape=jax.ShapeDtypeStruct((_N, _D), jnp.float32),
        interpret=interpret,
    )(nbr_t, u, w1a, w1b, b1, w2, b2, w3r)


def kernel(nodes, neighbor_nodes, emb, W1, b1, W2, b2, W3, b3):
    nbr_t_idx = jnp.pad(jnp.transpose(neighbor_nodes),
                        ((0, 0), (0, _NPAD - _N)))
    idx_nbr2 = nbr_t_idx.reshape(-1, 128)
    idx_u2 = jnp.pad(nodes, (0, _NPAD - _N)).reshape(-1, 128)
    nbr_flat, u = _sc_gather(emb, idx_nbr2, idx_u2)
    nbr_t = nbr_flat.reshape(_K, _NPAD, _D)
    return _tc_mlp(nbr_t, u, W1[:_D], W1[_D:], b1.reshape(1, _D),
                   W2, b2.reshape(1, _D), W3.reshape(1, _D))


# speedup vs baseline: 1.8876x; 1.0192x over previous
import functools

import jax
import jax.numpy as jnp
from jax import lax
from jax.experimental import pallas as pl
from jax.experimental.pallas import tpu as pltpu
from jax.experimental.pallas import tpu_sc as plsc

_N = 10000
_K = 32
_D = 128
_NPAD = 10240
_BN = 80
_NW = 32
_CHUNK_ROWS = 2
_CHUNK = _CHUNK_ROWS * 128


def _sc_gather(emb, idx_nbr2, idx_u2):
    mesh = plsc.VectorSubcoreMesh(core_axis_name="c", subcore_axis_name="s")
    nchunks = _NPAD // _CHUNK
    rows_per_w = _NPAD // 128

    @functools.partial(
        pl.kernel,
        out_type=(
            jax.ShapeDtypeStruct((_K * _NPAD, _D), jnp.float32),
            jax.ShapeDtypeStruct((_NPAD, _D), jnp.float32),
        ),
        mesh=mesh,
        scratch_types=[
            pltpu.VMEM((_CHUNK_ROWS, 128), jnp.int32),
            pltpu.VMEM((_CHUNK_ROWS, 128), jnp.int32),
            pltpu.VMEM((_CHUNK, _D), jnp.float32),
            pltpu.VMEM((_CHUNK, _D), jnp.float32),
            pltpu.SemaphoreType.DMA,
            pltpu.SemaphoreType.DMA,
        ],
    )
    def gather_kernel(emb_hbm, idxn_hbm, idxu_hbm, nbr_out, u_out,
                      idx_a, idx_b, rows_a, rows_b, sem_a, sem_b):
        wid = lax.axis_index("s") * 2 + lax.axis_index("c")
        base_row = wid * rows_per_w
        base_emb = wid * _NPAD

        def fire(c, idx_v, rows_v, sem):
            r0 = base_row + c * _CHUNK_ROWS
            pltpu.sync_copy(idxn_hbm.at[pl.ds(r0, _CHUNK_ROWS)], idx_v)
            for r in range(_CHUNK_ROWS):
                pltpu.async_copy(
                    emb_hbm.at[idx_v.at[r]],
                    rows_v.at[pl.ds(r * 128, 128)],
                    sem,
                )

        def drain_write(c, rows_v, sem):
            pltpu.make_async_copy(
                emb_hbm.at[pl.ds(0, _CHUNK)], rows_v, sem
            ).wait()
            pltpu.sync_copy(
                rows_v, nbr_out.at[pl.ds(base_emb + c * _CHUNK, _CHUNK)]
            )

        fire(0, idx_a, rows_a, sem_a)

        def pair(j, carry):
            c = j * 2
            fire(c + 1, idx_b, rows_b, sem_b)
            drain_write(c, rows_a, sem_a)

            @pl.when(c + 2 < nchunks)
            def _():
                fire(c + 2, idx_a, rows_a, sem_a)

            drain_write(c + 1, rows_b, sem_b)
            return carry

        lax.fori_loop(0, nchunks // 2, pair, 0)

        @pl.when(wid < 16)
        def _():
            def uchunk(c, carry):
                r0 = wid * 5 + c
                pltpu.sync_copy(idxu_hbm.at[pl.ds(r0, 1)],
                                idx_a.at[pl.ds(0, 1)])
                cp = pltpu.async_copy(
                    emb_hbm.at[idx_a.at[0]], rows_a.at[pl.ds(0, 128)], sem_a
                )
                cp.wait()
                pltpu.sync_copy(rows_a.at[pl.ds(0, 128)],
                                u_out.at[pl.ds(r0 * 128, 128)])
                return carry

            lax.fori_loop(0, 5, uchunk, 0)

    return gather_kernel(emb, idx_nbr2, idx_u2)


def _tc_body(nbr_ref, u_ref, w1a_ref, w1b_ref, b1_ref, w2_ref, b2_ref,
             w3_ref, out_ref):
    ub = u_ref[...]
    t = jnp.dot(ub, w1b_ref[...],
                preferred_element_type=jnp.float32) + b1_ref[...]
    nbr = nbr_ref[...]
    nbr_flat = nbr.reshape(_K * _BN, _D)
    h = jnp.dot(nbr_flat, w1a_ref[...], preferred_element_type=jnp.float32)
    h = h + jnp.broadcast_to(t[None], (_K, _BN, _D)).reshape(_K * _BN, _D)
    h = jnp.maximum(h, 0.0)
    h2 = jnp.dot(h, w2_ref[...], preferred_element_type=jnp.float32)
    h2 = jnp.maximum(h2 + b2_ref[...], 0.0)
    h2_3 = h2.reshape(_K, _BN, _D)
    w3v = w3_ref[...]
    s = [jnp.sum(h2_3[k] * w3v, axis=1, keepdims=True) for k in range(_K)]
    m = functools.reduce(jnp.maximum, s)
    e = [jnp.exp(sk - m) for sk in s]
    den = functools.reduce(lambda a, b: a + b, e)
    acc = e[0] * nbr[0]
    for k in range(1, _K):
        acc = acc + e[k] * nbr[k]
    out_ref[...] = acc / den


def _tc_mlp(nbr_t, u, w1a, w1b, b1, w2, b2, w3r, interpret=False):
    grid = _N // _BN
    return pl.pallas_call(
        _tc_body,
        grid=(grid,),
        in_specs=[
            pl.BlockSpec((_K, _BN, _D), lambda i: (0, i, 0)),
            pl.BlockSpec((_BN, _D), lambda i: (i, 0)),
            pl.BlockSpec((_D, _D), lambda i: (0, 0)),
            pl.BlockSpec((_D, _D), lambda i: (0, 0)),
            pl.BlockSpec((1, _D), lambda i: (0, 0)),
            pl.BlockSpec((_D, _D), lambda i: (0, 0)),
            pl.BlockSpec((1, _D), lambda i: (0, 0)),
            pl.BlockSpec((1, _D), lambda i: (0, 0)),
        ],
        out_specs=pl.BlockSpec((_BN, _D), lambda i: (i, 0)),
        out_shape=jax.ShapeDtypeStruct((_N, _D), jnp.float32),
        interpret=interpret,
    )(nbr_t, u, w1a, w1b, b1, w2, b2, w3r)


def kernel(nodes, neighbor_nodes, emb, W1, b1, W2, b2, W3, b3):
    nbr_t_idx = jnp.pad(jnp.transpose(neighbor_nodes),
                        ((0, 0), (0, _NPAD - _N)))
    idx_nbr2 = nbr_t_idx.reshape(-1, 128)
    idx_u2 = jnp.pad(nodes, (0, _NPAD - _N)).reshape(-1, 128)
    nbr_flat, u = _sc_gather(emb, idx_nbr2, idx_u2)
    nbr_t = nbr_flat.reshape(_K, _NPAD, _D)
    return _tc_mlp(nbr_t, u, W1[:_D], W1[_D:], b1.reshape(1, _D),
                   W2, b2.reshape(1, _D), W3.reshape(1, _D))


# speedup vs baseline: 1.9074x; 1.0105x over previous
import functools

import jax
import jax.numpy as jnp
from jax import lax
from jax.experimental import pallas as pl
from jax.experimental.pallas import tpu as pltpu
from jax.experimental.pallas import tpu_sc as plsc

_N = 10000
_K = 32
_D = 128
_NPAD = 10240
_BN = 80
_NW = 32
_CHUNK_ROWS = 2
_CHUNK = _CHUNK_ROWS * 128


def _sc_gather(emb, idx_nbr2, idx_u2):
    mesh = plsc.VectorSubcoreMesh(core_axis_name="c", subcore_axis_name="s")
    nchunks = _NPAD // _CHUNK
    rows_per_w = _NPAD // 128

    @functools.partial(
        pl.kernel,
        out_type=(
            jax.ShapeDtypeStruct((_K * _NPAD, _D), jnp.float32),
            jax.ShapeDtypeStruct((_NPAD, _D), jnp.float32),
        ),
        mesh=mesh,
        scratch_types=[
            pltpu.VMEM((_NPAD // 128, 1, 128), jnp.int32),
            pltpu.VMEM((5, 1, 128), jnp.int32),
            pltpu.VMEM((_CHUNK, _D), jnp.float32),
            pltpu.VMEM((_CHUNK, _D), jnp.float32),
            pltpu.SemaphoreType.DMA,
            pltpu.SemaphoreType.DMA,
        ],
    )
    def gather_kernel(emb_hbm, idxn_hbm, idxu_hbm, nbr_out, u_out,
                      idx_all, idx_u, rows_a, rows_b, sem_a, sem_b):
        wid = lax.axis_index("s") * 2 + lax.axis_index("c")
        base_row = wid * rows_per_w
        base_emb = wid * _NPAD

        pltpu.sync_copy(idxn_hbm.at[pl.ds(base_row, rows_per_w)], idx_all)

        def fire(c, rows_v, sem):
            for r in range(_CHUNK_ROWS):
                pltpu.async_copy(
                    emb_hbm.at[idx_all.at[c * _CHUNK_ROWS + r, 0]],
                    rows_v.at[pl.ds(r * 128, 128)],
                    sem,
                )

        def drain_write(c, rows_v, sem):
            pltpu.make_async_copy(
                emb_hbm.at[pl.ds(0, _CHUNK)], rows_v, sem
            ).wait()
            pltpu.sync_copy(
                rows_v, nbr_out.at[pl.ds(base_emb + c * _CHUNK, _CHUNK)]
            )

        fire(0, rows_a, sem_a)

        def pair(j, carry):
            c = j * 2
            fire(c + 1, rows_b, sem_b)
            drain_write(c, rows_a, sem_a)

            @pl.when(c + 2 < nchunks)
            def _():
                fire(c + 2, rows_a, sem_a)

            drain_write(c + 1, rows_b, sem_b)
            return carry

        lax.fori_loop(0, nchunks // 2, pair, 0)

        @pl.when(wid < 16)
        def _():
            pltpu.sync_copy(idxu_hbm.at[pl.ds(wid * 5, 5)], idx_u)

            def uchunk(c, carry):
                cp = pltpu.async_copy(
                    emb_hbm.at[idx_u.at[c, 0]], rows_a.at[pl.ds(0, 128)], sem_a
                )
                cp.wait()
                pltpu.sync_copy(rows_a.at[pl.ds(0, 128)],
                                u_out.at[pl.ds((wid * 5 + c) * 128, 128)])
                return carry

            lax.fori_loop(0, 5, uchunk, 0)

    return gather_kernel(emb, idx_nbr2, idx_u2)


def _tc_body(nbr_ref, u_ref, w1a_ref, w1b_ref, b1_ref, w2_ref, b2_ref,
             w3_ref, out_ref):
    ub = u_ref[...]
    t = jnp.dot(ub, w1b_ref[...],
                preferred_element_type=jnp.float32) + b1_ref[...]
    nbr = nbr_ref[...]
    nbr_flat = nbr.reshape(_K * _BN, _D)
    h = jnp.dot(nbr_flat, w1a_ref[...], preferred_element_type=jnp.float32)
    h = h + jnp.broadcast_to(t[None], (_K, _BN, _D)).reshape(_K * _BN, _D)
    h = jnp.maximum(h, 0.0)
    h2 = jnp.dot(h, w2_ref[...], preferred_element_type=jnp.float32)
    h2 = jnp.maximum(h2 + b2_ref[...], 0.0)
    h2_3 = h2.reshape(_K, _BN, _D)
    w3v = w3_ref[...]
    s = [jnp.sum(h2_3[k] * w3v, axis=1, keepdims=True) for k in range(_K)]
    m = functools.reduce(jnp.maximum, s)
    e = [jnp.exp(sk - m) for sk in s]
    den = functools.reduce(lambda a, b: a + b, e)
    acc = e[0] * nbr[0]
    for k in range(1, _K):
        acc = acc + e[k] * nbr[k]
    out_ref[...] = acc / den


def _tc_mlp(nbr_t, u, w1a, w1b, b1, w2, b2, w3r, interpret=False):
    grid = _N // _BN
    return pl.pallas_call(
        _tc_body,
        grid=(grid,),
        in_specs=[
            pl.BlockSpec((_K, _BN, _D), lambda i: (0, i, 0)),
            pl.BlockSpec((_BN, _D), lambda i: (i, 0)),
            pl.BlockSpec((_D, _D), lambda i: (0, 0)),
            pl.BlockSpec((_D, _D), lambda i: (0, 0)),
            pl.BlockSpec((1, _D), lambda i: (0, 0)),
            pl.BlockSpec((_D, _D), lambda i: (0, 0)),
            pl.BlockSpec((1, _D), lambda i: (0, 0)),
            pl.BlockSpec((1, _D), lambda i: (0, 0)),
        ],
        out_specs=pl.BlockSpec((_BN, _D), lambda i: (i, 0)),
        out_shape=jax.ShapeDtypeStruct((_N, _D), jnp.float32),
        interpret=interpret,
    )(nbr_t, u, w1a, w1b, b1, w2, b2, w3r)


def kernel(nodes, neighbor_nodes, emb, W1, b1, W2, b2, W3, b3):
    nbr_t_idx = jnp.pad(jnp.transpose(neighbor_nodes),
                        ((0, 0), (0, _NPAD - _N)))
    idx_nbr2 = nbr_t_idx.reshape(-1, 1, 128)
    idx_u2 = jnp.pad(nodes, (0, _NPAD - _N)).reshape(-1, 1, 128)
    nbr_flat, u = _sc_gather(emb, idx_nbr2, idx_u2)
    nbr_t = nbr_flat.reshape(_K, _NPAD, _D)
    return _tc_mlp(nbr_t, u, W1[:_D], W1[_D:], b1.reshape(1, _D),
                   W2, b2.reshape(1, _D), W3.reshape(1, _D))


# speedup vs baseline: 2.0723x; 1.0865x over previous
import functools

import jax
import jax.numpy as jnp
from jax import lax
from jax.experimental import pallas as pl
from jax.experimental.pallas import tpu as pltpu
from jax.experimental.pallas import tpu_sc as plsc

_N = 10000
_K = 32
_D = 128
_NPAD = 10240
_H = 4
_NC = _NPAD // _H
_BN = 80
_NW = 32
_CHUNK_ROWS = 2
_CHUNK = _CHUNK_ROWS * 128


def _sc_gather(emb, idx_nbr2, idx_u2):
    mesh = plsc.VectorSubcoreMesh(core_axis_name="c", subcore_axis_name="s")
    rows_per_w = _K * _NC // _NW // 128
    nchunks = rows_per_w // _CHUNK_ROWS
    u_rows = _NC // 128

    @functools.partial(
        pl.kernel,
        out_type=(
            jax.ShapeDtypeStruct((_K * _NC, _D), jnp.float32),
            jax.ShapeDtypeStruct((_NC, _D), jnp.float32),
        ),
        mesh=mesh,
        scratch_types=[
            pltpu.VMEM((rows_per_w, 1, 128), jnp.int32),
            pltpu.VMEM((1, 1, 128), jnp.int32),
            pltpu.VMEM((_CHUNK, _D), jnp.float32),
            pltpu.VMEM((_CHUNK, _D), jnp.float32),
            pltpu.SemaphoreType.DMA,
            pltpu.SemaphoreType.DMA,
        ],
    )
    def gather_kernel(emb_hbm, idxn_hbm, idxu_hbm, nbr_out, u_out,
                      idx_all, idx_u, rows_a, rows_b, sem_a, sem_b):
        wid = lax.axis_index("s") * 2 + lax.axis_index("c")
        base_row = wid * rows_per_w
        base_emb = wid * rows_per_w * 128

        pltpu.sync_copy(idxn_hbm.at[pl.ds(base_row, rows_per_w)], idx_all)

        def fire(c, rows_v, sem):
            for r in range(_CHUNK_ROWS):
                pltpu.async_copy(
                    emb_hbm.at[idx_all.at[c * _CHUNK_ROWS + r, 0]],
                    rows_v.at[pl.ds(r * 128, 128)],
                    sem,
                )

        def drain_write(c, rows_v, sem):
            pltpu.make_async_copy(
                emb_hbm.at[pl.ds(0, _CHUNK)], rows_v, sem
            ).wait()
            pltpu.sync_copy(
                rows_v, nbr_out.at[pl.ds(base_emb + c * _CHUNK, _CHUNK)]
            )

        fire(0, rows_a, sem_a)

        def pair(j, carry):
            c = j * 2
            fire(c + 1, rows_b, sem_b)
            drain_write(c, rows_a, sem_a)

            @pl.when(c + 2 < nchunks)
            def _():
                fire(c + 2, rows_a, sem_a)

            drain_write(c + 1, rows_b, sem_b)
            return carry

        lax.fori_loop(0, nchunks // 2, pair, 0)

        @pl.when(wid < u_rows)
        def _():
            pltpu.sync_copy(idxu_hbm.at[pl.ds(wid, 1)], idx_u)
            cp = pltpu.async_copy(
                emb_hbm.at[idx_u.at[0, 0]], rows_a.at[pl.ds(0, 128)], sem_a
            )
            cp.wait()
            pltpu.sync_copy(rows_a.at[pl.ds(0, 128)],
                            u_out.at[pl.ds(wid * 128, 128)])

    return gather_kernel(emb, idx_nbr2, idx_u2)


def _tc_body(nbr_ref, u_ref, w1a_ref, w1b_ref, b1_ref, w2_ref, b2_ref,
             w3_ref, out_ref):
    ub = u_ref[...]
    t = jnp.dot(ub, w1b_ref[...],
                preferred_element_type=jnp.float32) + b1_ref[...]
    nbr = nbr_ref[...]
    nbr_flat = nbr.reshape(_K * _BN, _D)
    h = jnp.dot(nbr_flat, w1a_ref[...], preferred_element_type=jnp.float32)
    h = h + jnp.broadcast_to(t[None], (_K, _BN, _D)).reshape(_K * _BN, _D)
    h = jnp.maximum(h, 0.0)
    h2 = jnp.dot(h, w2_ref[...], preferred_element_type=jnp.float32)
    h2 = jnp.maximum(h2 + b2_ref[...], 0.0)
    h2_3 = h2.reshape(_K, _BN, _D)
    w3v = w3_ref[...]
    s = [jnp.sum(h2_3[k] * w3v, axis=1, keepdims=True) for k in range(_K)]
    m = functools.reduce(jnp.maximum, s)
    e = [jnp.exp(sk - m) for sk in s]
    den = functools.reduce(lambda a, b: a + b, e)
    acc = e[0] * nbr[0]
    for k in range(1, _K):
        acc = acc + e[k] * nbr[k]
    out_ref[...] = acc / den


def _tc_mlp(nbr_t, u, w1a, w1b, b1, w2, b2, w3r, interpret=False):
    nc = nbr_t.shape[1]
    return pl.pallas_call(
        _tc_body,
        grid=(nc // _BN,),
        in_specs=[
            pl.BlockSpec((_K, _BN, _D), lambda i: (0, i, 0)),
            pl.BlockSpec((_BN, _D), lambda i: (i, 0)),
            pl.BlockSpec((_D, _D), lambda i: (0, 0)),
            pl.BlockSpec((_D, _D), lambda i: (0, 0)),
            pl.BlockSpec((1, _D), lambda i: (0, 0)),
            pl.BlockSpec((_D, _D), lambda i: (0, 0)),
            pl.BlockSpec((1, _D), lambda i: (0, 0)),
            pl.BlockSpec((1, _D), lambda i: (0, 0)),
        ],
        out_specs=pl.BlockSpec((_BN, _D), lambda i: (i, 0)),
        out_shape=jax.ShapeDtypeStruct((nc, _D), jnp.float32),
        interpret=interpret,
    )(nbr_t, u, w1a, w1b, b1, w2, b2, w3r)


def kernel(nodes, neighbor_nodes, emb, W1, b1, W2, b2, W3, b3):
    nbr_pad = jnp.pad(neighbor_nodes, ((0, _NPAD - _N), (0, 0)))
    nbr_t_idx = jnp.transpose(nbr_pad.reshape(_H, _NC, _K), (0, 2, 1))
    idx_nbr = nbr_t_idx.reshape(_H, -1, 1, 128)
    idx_u = jnp.pad(nodes, (0, _NPAD - _N)).reshape(_H, -1, 1, 128)
    w1a, w1b = W1[:_D], W1[_D:]
    b1r, b2r, w3r = b1.reshape(1, _D), b2.reshape(1, _D), W3.reshape(1, _D)
    outs = []
    for h in range(_H):
        nbr_flat, u = _sc_gather(emb, idx_nbr[h], idx_u[h])
        outs.append(_tc_mlp(nbr_flat.reshape(_K, _NC, _D), u,
                            w1a, w1b, b1r, W2, b2r, w3r))
    return jnp.concatenate(outs, axis=0)[:_N]


# speedup vs baseline: 4.2540x; 2.0527x over previous
import functools

import jax
import jax.numpy as jnp
from jax import lax
from jax.experimental import pallas as pl
from jax.experimental.pallas import tpu as pltpu
from jax.experimental.pallas import tpu_sc as plsc

_N = 10000
_K = 32
_D = 128
_NPAD = 10240
_H = 4
_NC = _NPAD // _H
_BN = 80
_NW = 32
_CHUNK_ROWS = 2
_CHUNK = _CHUNK_ROWS * 128


def _sc_gather(emb, idx_nbr2, idx_u2):
    mesh = plsc.VectorSubcoreMesh(core_axis_name="c", subcore_axis_name="s")
    rows_per_w = _K * _NC // _NW // 128
    nchunks = rows_per_w // _CHUNK_ROWS
    u_rows = _NC // 128

    @functools.partial(
        pl.kernel,
        out_type=(
            jax.ShapeDtypeStruct((_K * _NC, _D), jnp.float32),
            jax.ShapeDtypeStruct((_NC, _D), jnp.float32),
        ),
        mesh=mesh,
        scratch_types=[
            pltpu.VMEM((rows_per_w, 1, 128), jnp.int32),
            pltpu.VMEM((1, 1, 128), jnp.int32),
            pltpu.VMEM((_CHUNK, _D), jnp.float32),
            pltpu.VMEM((_CHUNK, _D), jnp.float32),
            pltpu.SemaphoreType.DMA,
            pltpu.SemaphoreType.DMA,
        ],
    )
    def gather_kernel(emb_hbm, idxn_hbm, idxu_hbm, nbr_out, u_out,
                      idx_all, idx_u, rows_a, rows_b, sem_a, sem_b):
        wid = lax.axis_index("s") * 2 + lax.axis_index("c")
        base_row = wid * rows_per_w
        base_emb = wid * rows_per_w * 128

        pltpu.sync_copy(idxn_hbm.at[pl.ds(base_row, rows_per_w)], idx_all)

        def fire(c, rows_v, sem):
            for r in range(_CHUNK_ROWS):
                pltpu.async_copy(
                    emb_hbm.at[idx_all.at[c * _CHUNK_ROWS + r, 0]],
                    rows_v.at[pl.ds(r * 128, 128)],
                    sem,
                )

        def drain_write(c, rows_v, sem):
            pltpu.make_async_copy(
                emb_hbm.at[pl.ds(0, _CHUNK)], rows_v, sem
            ).wait()
            pltpu.sync_copy(
                rows_v, nbr_out.at[pl.ds(base_emb + c * _CHUNK, _CHUNK)]
            )

        fire(0, rows_a, sem_a)

        def pair(j, carry):
            c = j * 2
            fire(c + 1, rows_b, sem_b)
            drain_write(c, rows_a, sem_a)

            @pl.when(c + 2 < nchunks)
            def _():
                fire(c + 2, rows_a, sem_a)

            drain_write(c + 1, rows_b, sem_b)
            return carry

        lax.fori_loop(0, nchunks // 2, pair, 0)

        @pl.when(wid < u_rows)
        def _():
            pltpu.sync_copy(idxu_hbm.at[pl.ds(wid, 1)], idx_u)
            cp = pltpu.async_copy(
                emb_hbm.at[idx_u.at[0, 0]], rows_a.at[pl.ds(0, 128)], sem_a
            )
            cp.wait()
            pltpu.sync_copy(rows_a.at[pl.ds(0, 128)],
                            u_out.at[pl.ds(wid * 128, 128)])

    return gather_kernel(emb, idx_nbr2, idx_u2)


def _tc_body(nbr_ref, u_ref, w1a_ref, w1b_ref, b1_ref, w2_ref, b2_ref,
             w3_ref, out_ref):
    ub = u_ref[...]
    t = jnp.dot(ub, w1b_ref[...],
                preferred_element_type=jnp.float32) + b1_ref[...]
    nbr = nbr_ref[...]
    nbr_flat = nbr.reshape(_K * _BN, _D)
    h = jnp.dot(nbr_flat, w1a_ref[...], preferred_element_type=jnp.float32)
    h = h + jnp.broadcast_to(t[None], (_K, _BN, _D)).reshape(_K * _BN, _D)
    h = jnp.maximum(h, 0.0)
    h2 = jnp.dot(h, w2_ref[...], preferred_element_type=jnp.float32)
    h2 = jnp.maximum(h2 + b2_ref[...], 0.0)
    h2_3 = h2.reshape(_K, _BN, _D)
    w3v = w3_ref[...]
    s = [jnp.sum(h2_3[k] * w3v, axis=1, keepdims=True) for k in range(_K)]
    m = functools.reduce(jnp.maximum, s)
    e = [jnp.exp(sk - m) for sk in s]
    den = functools.reduce(lambda a, b: a + b, e)
    acc = e[0] * nbr[0]
    for k in range(1, _K):
        acc = acc + e[k] * nbr[k]
    out_ref[...] = acc / den


def _tc_mlp(nbr_t, u, w1a, w1b, b1, w2, b2, w3r, interpret=False):
    nc = nbr_t.shape[1]
    return pl.pallas_call(
        _tc_body,
        grid=(nc // _BN,),
        in_specs=[
            pl.BlockSpec((_K, _BN, _D), lambda i: (0, i, 0)),
            pl.BlockSpec((_BN, _D), lambda i: (i, 0)),
            pl.BlockSpec((_D, _D), lambda i: (0, 0)),
            pl.BlockSpec((_D, _D), lambda i: (0, 0)),
            pl.BlockSpec((1, _D), lambda i: (0, 0)),
            pl.BlockSpec((_D, _D), lambda i: (0, 0)),
            pl.BlockSpec((1, _D), lambda i: (0, 0)),
            pl.BlockSpec((1, _D), lambda i: (0, 0)),
        ],
        out_specs=pl.BlockSpec((_BN, _D), lambda i: (i, 0)),
        out_shape=jax.ShapeDtypeStruct((nc, _D), jnp.float32),
        interpret=interpret,
    )(nbr_t, u, w1a, w1b, b1, w2, b2, w3r)


def kernel(nodes, neighbor_nodes, emb, W1, b1, W2, b2, W3, b3):
    pad = jnp.arange(_NPAD - _N, dtype=neighbor_nodes.dtype)
    nbr_pad = jnp.concatenate(
        [neighbor_nodes, jnp.broadcast_to(pad[:, None], (_NPAD - _N, _K))],
        axis=0)
    nbr_t_idx = jnp.transpose(nbr_pad.reshape(_H, _NC, _K), (0, 2, 1))
    idx_nbr = nbr_t_idx.reshape(_H, -1, 1, 128)
    idx_u = jnp.concatenate([nodes, pad]).reshape(_H, -1, 1, 128)
    w1a, w1b = W1[:_D], W1[_D:]
    b1r, b2r, w3r = b1.reshape(1, _D), b2.reshape(1, _D), W3.reshape(1, _D)
    outs = []
    for h in range(_H):
        nbr_flat, u = _sc_gather(emb, idx_nbr[h], idx_u[h])
        outs.append(_tc_mlp(nbr_flat.reshape(_K, _NC, _D), u,
                            w1a, w1b, b1r, W2, b2r, w3r))
    return jnp.concatenate(outs, axis=0)[:_N]
